# baseline (speedup 1.0000x reference)
import functools

import numpy as np
import jax
import jax.numpy as jnp
from jax import lax
from jax.experimental import pallas as pl
from jax.experimental.pallas import tpu as pltpu
from jax.experimental.pallas import tpu_sc as plsc

NUM_RELATIONS = 100000
EMB_DIM = 64
BATCH = 16384

NC = 2
NS = 16
LANES = 16
NW = NC * NS
BPW = BATCH // NW
CHUNK = 128
NCHUNK = BPW // CHUNK
GROUPS = CHUNK // LANES
DSUB = EMB_DIM // LANES

def _lane_sum(v, shuf_idx):
    for idx in shuf_idx:
        v = v + jnp.take_along_axis(v, idx, axis=0,
                                    mode="promise_in_bounds")
    return v


def _sc_body(xi_hbm, xj_hbm, idx_hbm, tab_hbm, out_hbm,
             idx_v, rel_v, xi_v, xj_v, out_v, sem):
    wid = lax.axis_index("s") * NC + lax.axis_index("c")
    base = wid * BPW
    lane = lax.iota(jnp.int32, LANES)
    shuf_idx = [lane ^ s for s in (8, 4, 2, 1)]

    pltpu.sync_copy(idx_hbm.at[pl.ds(wid * NCHUNK, NCHUNK)], idx_v)

    for c in range(NCHUNK):
        cbase = base + c * CHUNK
        gather = pltpu.async_copy(tab_hbm.at[idx_v.at[c]], rel_v, sem)
        pltpu.sync_copy(xi_hbm.at[pl.ds(cbase, CHUNK)], xi_v)
        pltpu.sync_copy(xj_hbm.at[pl.ds(cbase, CHUNK)], xj_v)
        gather.wait()

        def group(g, carry, c=c):
            acc = jnp.zeros((LANES,), jnp.float32)
            for r in range(LANES):
                row = g * LANES + r
                p = jnp.zeros((LANES,), jnp.float32)
                for k in range(DSUB):
                    a = xi_v[row, pl.ds(k * LANES, LANES)]
                    t = rel_v[row, pl.ds(k * LANES, LANES)]
                    b = xj_v[row, pl.ds(k * LANES, LANES)]
                    p = p + a * t * b
                tot = _lane_sum(p, shuf_idx)
                acc = jnp.where(lane == r, tot, acc)
            out_v[pl.ds(c * CHUNK + g * LANES, LANES)] = acc
            return carry

        lax.fori_loop(0, GROUPS, group, 0)

    pltpu.sync_copy(out_v, out_hbm.at[pl.ds(base, BPW)])


@jax.jit
def _run(x_i, x_j, idx2d, relation_embedding):
    mesh = plsc.VectorSubcoreMesh(core_axis_name="c", subcore_axis_name="s")
    return pl.kernel(
        _sc_body,
        out_type=jax.ShapeDtypeStruct((BATCH,), jnp.float32),
        mesh=mesh,
        compiler_params=pltpu.CompilerParams(use_tc_tiling_on_sc=False),
        scratch_types=[
            pltpu.VMEM((NCHUNK, CHUNK), jnp.int32),
            pltpu.VMEM((CHUNK, EMB_DIM), jnp.float32),
            pltpu.VMEM((CHUNK, EMB_DIM), jnp.float32),
            pltpu.VMEM((CHUNK, EMB_DIM), jnp.float32),
            pltpu.VMEM((BPW,), jnp.float32),
            pltpu.SemaphoreType.DMA,
        ],
    )(x_i, x_j, idx2d, relation_embedding)


def kernel(x_i, x_j, edge_type, relation_embedding):
    idx2d = edge_type.astype(jnp.int32).reshape(NW * NCHUNK, CHUNK)
    return _run(x_i, x_j, idx2d, relation_embedding)


# speedup vs baseline: 1.1502x; 1.1502x over previous
import functools

import jax
import jax.numpy as jnp
from jax import lax
from jax.experimental import pallas as pl
from jax.experimental.pallas import tpu as pltpu
from jax.experimental.pallas import tpu_sc as plsc

NUM_RELATIONS = 100000
EMB_DIM = 64
BATCH = 16384

NC = 2
NS = 16
LANES = 16
NW = NC * NS
BPW = BATCH // NW
CHUNK = 128
NCHUNK = BPW // CHUNK
GROUPS = CHUNK // LANES


HALF = BPW // 2
HCHUNK = HALF // CHUNK
HGROUPS = HALF // LANES


def _sc_body(xi_hbm, xj_hbm, idx_hbm, tab_hbm, out_hbm,
             idx_v, pair_v, rel_v, xi_v, xj_v, out_v, sem):
    wid = lax.axis_index("s") * NC + lax.axis_index("c")
    base = wid * BPW

    pltpu.sync_copy(idx_hbm.at[pl.ds(wid * NCHUNK, NCHUNK)], idx_v)
    for c in range(NCHUNK):
        for k in range(CHUNK // LANES):
            e = idx_v[c, pl.ds(k * LANES, LANES)]
            pair_v[c, pl.ds(k * LANES, LANES)] = lax.shift_right_logical(e, 1)

    for h in range(2):
        hbase = base + h * HALF
        cp_xi = pltpu.async_copy(xi_hbm.at[:, pl.ds(hbase, HALF)], xi_v, sem)
        cp_xj = pltpu.async_copy(xj_hbm.at[:, pl.ds(hbase, HALF)], xj_v, sem)
        gathers = [
            pltpu.async_copy(tab_hbm.at[pair_v.at[h * HCHUNK + c]],
                             rel_v.at[pl.ds(c * CHUNK, CHUNK)], sem)
            for c in range(HCHUNK)
        ]
        cp_xi.wait()
        cp_xj.wait()
        for g in gathers:
            g.wait()

        def group(g, carry, h=h):
            ebase = g * LANES
            lane = lax.iota(jnp.int32, LANES)
            rowv = lane + ebase
            par = jnp.bitwise_and(
                idx_v[h * HCHUNK + g // GROUPS,
                      pl.ds((g % GROUPS) * LANES, LANES)], 1)
            colbase = par * EMB_DIM
            acc = jnp.zeros((LANES,), jnp.float32)
            for d in range(EMB_DIM):
                r = plsc.load_gather(rel_v, [rowv, colbase + d])
                a = xi_v[d, pl.ds(ebase, LANES)]
                b = xj_v[d, pl.ds(ebase, LANES)]
                acc = acc + a * r * b
            out_v[pl.ds(h * HALF + ebase, LANES)] = acc
            return carry

        lax.fori_loop(0, HGROUPS, group, 0)

    pltpu.sync_copy(out_v, out_hbm.at[pl.ds(base, BPW)])


@jax.jit
def _run(xt_i, xt_j, idx2d, tab2):
    mesh = plsc.VectorSubcoreMesh(core_axis_name="c", subcore_axis_name="s")
    return pl.kernel(
        _sc_body,
        out_type=jax.ShapeDtypeStruct((BATCH,), jnp.float32),
        mesh=mesh,
        compiler_params=pltpu.CompilerParams(needs_layout_passes=False),
        scratch_types=[
            pltpu.VMEM((NCHUNK, CHUNK), jnp.int32),
            pltpu.VMEM((NCHUNK, CHUNK), jnp.int32),
            pltpu.VMEM((HALF, 2 * EMB_DIM), jnp.float32),
            pltpu.VMEM((EMB_DIM, HALF), jnp.float32),
            pltpu.VMEM((EMB_DIM, HALF), jnp.float32),
            pltpu.VMEM((BPW,), jnp.float32),
            pltpu.SemaphoreType.DMA,
        ],
    )(xt_i, xt_j, idx2d, tab2)


def kernel(x_i, x_j, edge_type, relation_embedding):
    idx2d = edge_type.astype(jnp.int32).reshape(NW * NCHUNK, CHUNK)
    tab2 = relation_embedding.reshape(NUM_RELATIONS // 2, 2 * EMB_DIM)
    return _run(x_i.T, x_j.T, idx2d, tab2)
